# baseline (speedup 1.0000x reference)
import functools

import jax
import jax.numpy as jnp
from jax import lax
from jax.experimental import pallas as pl
from jax.experimental.pallas import tpu as pltpu
from jax.experimental.pallas import tpu_sc as plsc

N = 10000
NPAD = 10240
E = 320000
NFEAT = 128
HID = 128
NCLASS = 64
LAMDA = 0.5

NC = 2
NS = 16
NW = NC * NS
CHUNK = 128
NCHUNK = 80
EPP = NCHUNK * CHUNK
EPAD = NW * EPP
NPAIR = NCHUNK // 2
ROWBLK = 2048
NBLK = NPAD // ROWBLK


def _k1_body(x_ref, we_ref, be_ref, w0_ref, b0_ref, out_ref):
    wc = jnp.dot(we_ref[...], w0_ref[...], preferred_element_type=jnp.float32)
    bc = jnp.dot(be_ref[...], w0_ref[...], preferred_element_type=jnp.float32) + b0_ref[...]
    out_ref[...] = jnp.dot(x_ref[...], wc, preferred_element_type=jnp.float32) + bc


def _k1(xs, W_enc, b_enc, W0, b0):
    return pl.pallas_call(
        _k1_body,
        grid=(NBLK,),
        in_specs=[
            pl.BlockSpec((ROWBLK, NFEAT), lambda i: (i, 0)),
            pl.BlockSpec((NFEAT, HID), lambda i: (0, 0)),
            pl.BlockSpec((1, HID), lambda i: (0, 0)),
            pl.BlockSpec((HID, HID), lambda i: (0, 0)),
            pl.BlockSpec((1, HID), lambda i: (0, 0)),
        ],
        out_specs=pl.BlockSpec((ROWBLK, HID), lambda i: (i, 0)),
        out_shape=jax.ShapeDtypeStruct((NPAD, HID), jnp.float32),
    )(xs, W_enc, b_enc, W0, b0)


STRIPE = NPAD // NS


def _k2_body(dst_hbm, zeros_hbm, out_hbm, dstv, hist, gath, sdeg):
    c = lax.axis_index("c")
    s = lax.axis_index("s")
    wid = c * NS + s
    pltpu.sync_copy(zeros_hbm, hist)
    pltpu.sync_copy(dst_hbm.at[wid], dstv)
    ones_v = jnp.ones((16,), jnp.float32)

    def step(i, _):
        d = dstv[i]
        plsc.addupdate_scatter(hist, [d], ones_v)
        return 0

    lax.fori_loop(0, EPP // 16, step, 0)
    pltpu.sync_copy(hist, sdeg.at[s])
    plsc.subcore_barrier()
    pltpu.sync_copy(sdeg.at[:, pl.ds(s * STRIPE, STRIPE)], gath)

    def red(j, _):
        sl = pl.ds(j * 16, 16)
        v = gath[0, sl]
        for t in range(1, NS):
            v = v + gath[t, sl]
        hist[sl] = v
        return 0

    lax.fori_loop(0, STRIPE // 16, red, 0)
    pltpu.sync_copy(hist.at[pl.ds(0, STRIPE)], out_hbm.at[c, pl.ds(s * STRIPE, STRIPE)])


def _k2(dst2, zeros16):
    mesh = plsc.VectorSubcoreMesh(core_axis_name="c", subcore_axis_name="s")
    kfn = pl.kernel(
        _k2_body,
        out_type=jax.ShapeDtypeStruct((NC, NPAD), jnp.float32),
        mesh=mesh,
        scratch_types=[
            pltpu.VMEM((EPP // 16, 16), jnp.int32),
            pltpu.VMEM((NPAD,), jnp.float32),
            pltpu.VMEM((NS, STRIPE), jnp.float32),
            pltpu.VMEM_SHARED((NS, NPAD), jnp.float32),
        ],
        compiler_params=pltpu.CompilerParams(needs_layout_passes=False),
    )
    return kfn(dst2, zeros16)


def _k3_body(degp_ref, zero_ref, dinv_ref, g_ref):
    deg = jnp.sum(degp_ref[...], axis=0)
    dv = lax.rsqrt(jnp.maximum(deg, 1.0))
    dinv_ref[...] = dv
    g_ref[...] = dv * zero_ref[...]


def _k3(degp, zero_):
    return pl.pallas_call(
        _k3_body,
        grid=(NBLK,),
        in_specs=[
            pl.BlockSpec((NC, ROWBLK, 1), lambda i: (0, i, 0)),
            pl.BlockSpec((ROWBLK, HID), lambda i: (i, 0)),
        ],
        out_specs=[
            pl.BlockSpec((ROWBLK, 1), lambda i: (i, 0)),
            pl.BlockSpec((ROWBLK, HID), lambda i: (i, 0)),
        ],
        out_shape=[
            jax.ShapeDtypeStruct((NPAD, 1), jnp.float32),
            jax.ShapeDtypeStruct((NPAD, HID), jnp.float32),
        ],
    )(degp, zero_)


def _k4_body(src_hbm, dst_hbm, g_hbm, zeros_hbm, out_hbm,
             dstidx, ring0, ring1, rows0, rows1,
             sem0, sem1, semi0, semi1, sems0, sems1, acc):
    c = lax.axis_index("c")
    s = lax.axis_index("s")
    wid = c * NS + s
    pltpu.sync_copy(zeros_hbm, acc.at[pl.ds(s * (NPAD // NS), NPAD // NS)])
    pltpu.sync_copy(dst_hbm.at[wid], dstidx)
    plsc.subcore_barrier()

    pltpu.sync_copy(src_hbm.at[wid, 0], ring0)
    pltpu.async_copy(g_hbm.at[ring0], rows0, sem0)
    pltpu.async_copy(src_hbm.at[wid, 1], ring1, semi1)
    pltpu.make_async_copy(src_hbm.at[wid, 0], ring1, semi1).wait()
    pltpu.async_copy(g_hbm.at[ring1], rows1, sem1)
    last = NCHUNK - 1

    def pair(p, _):
        i0 = 2 * p
        pltpu.make_async_copy(g_hbm.at[ring0], rows0, sem0).wait()
        pltpu.async_copy(rows0, acc.at[dstidx.at[i0]], sems0, add=True)
        pltpu.async_copy(src_hbm.at[wid, jnp.minimum(i0 + 2, last)], ring0, semi0)
        pltpu.make_async_copy(g_hbm.at[ring1], rows1, sem1).wait()
        pltpu.async_copy(rows1, acc.at[dstidx.at[i0 + 1]], sems1, add=True)
        pltpu.async_copy(src_hbm.at[wid, jnp.minimum(i0 + 3, last)], ring1, semi1)

        @pl.when(i0 + 2 < NCHUNK)
        def _():
            pltpu.make_async_copy(rows0, acc.at[dstidx.at[i0]], sems0).wait()
            pltpu.make_async_copy(src_hbm.at[wid, 0], ring0, semi0).wait()
            pltpu.async_copy(g_hbm.at[ring0], rows0, sem0)

        @pl.when(i0 + 3 < NCHUNK)
        def _():
            pltpu.make_async_copy(rows1, acc.at[dstidx.at[i0 + 1]], sems1).wait()
            pltpu.make_async_copy(src_hbm.at[wid, 0], ring1, semi1).wait()
            pltpu.async_copy(g_hbm.at[ring1], rows1, sem1)

        return 0

    lax.fori_loop(0, NPAIR, pair, 0)
    pltpu.make_async_copy(rows0, acc.at[dstidx.at[0]], sems0).wait()
    pltpu.make_async_copy(rows1, acc.at[dstidx.at[0]], sems1).wait()
    pltpu.make_async_copy(src_hbm.at[wid, 0], ring0, semi0).wait()
    pltpu.make_async_copy(src_hbm.at[wid, 0], ring1, semi1).wait()
    plsc.subcore_barrier()
    pltpu.sync_copy(acc.at[pl.ds(s * (NPAD // NS), NPAD // NS)], out_hbm.at[c, s])


def _k4(src2, dst2, g, zeros640):
    mesh = plsc.VectorSubcoreMesh(core_axis_name="c", subcore_axis_name="s")
    kfn = pl.kernel(
        _k4_body,
        out_type=jax.ShapeDtypeStruct((NC, NS, NPAD // NS, HID), jnp.float32),
        mesh=mesh,
        scratch_types=[
            pltpu.VMEM((NCHUNK, CHUNK), jnp.int32),
            pltpu.VMEM((CHUNK,), jnp.int32),
            pltpu.VMEM((CHUNK,), jnp.int32),
            pltpu.VMEM((CHUNK, HID), jnp.float32),
            pltpu.VMEM((CHUNK, HID), jnp.float32),
            pltpu.SemaphoreType.DMA,
            pltpu.SemaphoreType.DMA,
            pltpu.SemaphoreType.DMA,
            pltpu.SemaphoreType.DMA,
            pltpu.SemaphoreType.DMA,
            pltpu.SemaphoreType.DMA,
            pltpu.VMEM_SHARED((NPAD, HID), jnp.float32),
        ],
        compiler_params=pltpu.CompilerParams(needs_layout_passes=False),
    )
    return kfn(src2, dst2, g, zeros640)


def _k5_body(accp_ref, dinv_ref, zero_ref, w1_ref, b1_ref, out_ref):
    acc = accp_ref[0] + accp_ref[1]
    dv = dinv_ref[...]
    beta = LAMDA
    layer = (1.0 - beta) * (dv * acc) + beta * zero_ref[...]
    f = jnp.dot(layer, w1_ref[...], preferred_element_type=jnp.float32) + b1_ref[...]
    m = jnp.max(f, axis=1, keepdims=True)
    lse = jnp.log(jnp.sum(jnp.exp(f - m), axis=1, keepdims=True)) + m
    out_ref[...] = f - lse


def _k5(accp, dinv, zero_, W1, b1):
    return pl.pallas_call(
        _k5_body,
        grid=(NBLK,),
        in_specs=[
            pl.BlockSpec((NC, ROWBLK, HID), lambda i: (0, i, 0)),
            pl.BlockSpec((ROWBLK, 1), lambda i: (i, 0)),
            pl.BlockSpec((ROWBLK, HID), lambda i: (i, 0)),
            pl.BlockSpec((HID, NCLASS), lambda i: (0, 0)),
            pl.BlockSpec((1, NCLASS), lambda i: (0, 0)),
        ],
        out_specs=pl.BlockSpec((ROWBLK, NCLASS), lambda i: (i, 0)),
        out_shape=jax.ShapeDtypeStruct((NPAD, NCLASS), jnp.float32),
    )(accp, dinv, zero_, W1, b1)


@jax.jit
def kernel(x, edge_index, W_enc, b_enc, W0, b0, W1, b1):
    xs = jnp.squeeze(x, 0)
    xs = jnp.pad(xs, ((0, NPAD - N), (0, 0)))
    pad_idx = N + (jnp.arange(EPAD - E, dtype=jnp.int32) % (NPAD - N))
    ei = jnp.concatenate([edge_index, jnp.stack([pad_idx, pad_idx])], axis=1)
    src2 = ei[0].reshape(NW, NCHUNK, CHUNK)
    dst2 = ei[1].reshape(NW, NCHUNK, CHUNK)
    dst2h = ei[1].reshape(NW, EPP // 16, 16)
    b_enc2 = b_enc.reshape(1, HID)
    b02 = b0.reshape(1, HID)
    b12 = b1.reshape(1, NCLASS)

    zero_ = _k1(xs, W_enc, b_enc2, W0, b02)

    zeros16 = jnp.zeros((NPAD,), jnp.float32)
    degp = _k2(dst2h, zeros16)
    degp = degp.reshape(NC, NPAD, 1)

    dinv, g = _k3(degp, zero_)

    zeros640 = jnp.zeros((NPAD // NS, HID), jnp.float32)
    accp = _k4(src2, dst2, g, zeros640)
    accp = accp.reshape(NC, NPAD, HID)

    logp = _k5(accp, dinv, zero_, W1, b12)[:N]
    return (logp, jnp.float32(0.0), 0, 0)


# speedup vs baseline: 1.0818x; 1.0818x over previous
import functools

import jax
import jax.numpy as jnp
from jax import lax
from jax.experimental import pallas as pl
from jax.experimental.pallas import tpu as pltpu
from jax.experimental.pallas import tpu_sc as plsc

N = 10000
NPAD = 10240
E = 320000
NFEAT = 128
HID = 128
NCLASS = 64
LAMDA = 0.5

NC = 2
NS = 16
NW = NC * NS
CHUNK = 128
NCHUNK = 80
EPP = NCHUNK * CHUNK
EPAD = NW * EPP
NPAIR = NCHUNK // 2
ROWBLK = 2048
NBLK = NPAD // ROWBLK


STRIPE = NPAD // NS


def _k2_body(dst_hbm, zeros_hbm, out_hbm, dstv, hist, gath, sdeg):
    c = lax.axis_index("c")
    s = lax.axis_index("s")
    wid = c * NS + s
    pltpu.sync_copy(zeros_hbm, hist)
    pltpu.sync_copy(dst_hbm.at[wid], dstv)
    ones_v = jnp.ones((16,), jnp.float32)

    def step(i, _):
        d = dstv[i]
        plsc.addupdate_scatter(hist, [d], ones_v)
        return 0

    lax.fori_loop(0, EPP // 16, step, 0)
    pltpu.sync_copy(hist, sdeg.at[s])
    plsc.subcore_barrier()
    pltpu.sync_copy(sdeg.at[:, pl.ds(s * STRIPE, STRIPE)], gath)

    def red(j, _):
        sl = pl.ds(j * 16, 16)
        v = gath[0, sl]
        for t in range(1, NS):
            v = v + gath[t, sl]
        hist[sl] = v
        return 0

    lax.fori_loop(0, STRIPE // 16, red, 0)
    pltpu.sync_copy(hist.at[pl.ds(0, STRIPE)], out_hbm.at[c, pl.ds(s * STRIPE, STRIPE)])


def _k2(dst2, zeros16):
    mesh = plsc.VectorSubcoreMesh(core_axis_name="c", subcore_axis_name="s")
    kfn = pl.kernel(
        _k2_body,
        out_type=jax.ShapeDtypeStruct((NC, NPAD), jnp.float32),
        mesh=mesh,
        scratch_types=[
            pltpu.VMEM((EPP // 16, 16), jnp.int32),
            pltpu.VMEM((NPAD,), jnp.float32),
            pltpu.VMEM((NS, STRIPE), jnp.float32),
            pltpu.VMEM_SHARED((NS, NPAD), jnp.float32),
        ],
        compiler_params=pltpu.CompilerParams(needs_layout_passes=False),
    )
    return kfn(dst2, zeros16)


def _k3_body(x_ref, we_ref, be_ref, w0_ref, b0_ref, degp_ref,
             zero_ref, dinv_ref, g_ref):
    wc = jnp.dot(we_ref[...], w0_ref[...], preferred_element_type=jnp.float32)
    bc = jnp.dot(be_ref[...], w0_ref[...], preferred_element_type=jnp.float32) + b0_ref[...]
    zero = jnp.dot(x_ref[...], wc, preferred_element_type=jnp.float32) + bc
    zero_ref[...] = zero
    deg = jnp.sum(degp_ref[...], axis=0)
    dv = lax.rsqrt(jnp.maximum(deg, 1.0))
    dinv_ref[...] = dv
    g_ref[...] = dv * zero


def _k3(xs, W_enc, b_enc, W0, b0, degp):
    return pl.pallas_call(
        _k3_body,
        grid=(NBLK,),
        in_specs=[
            pl.BlockSpec((ROWBLK, NFEAT), lambda i: (i, 0)),
            pl.BlockSpec((NFEAT, HID), lambda i: (0, 0)),
            pl.BlockSpec((1, HID), lambda i: (0, 0)),
            pl.BlockSpec((HID, HID), lambda i: (0, 0)),
            pl.BlockSpec((1, HID), lambda i: (0, 0)),
            pl.BlockSpec((NC, ROWBLK, 1), lambda i: (0, i, 0)),
        ],
        out_specs=[
            pl.BlockSpec((ROWBLK, HID), lambda i: (i, 0)),
            pl.BlockSpec((ROWBLK, 1), lambda i: (i, 0)),
            pl.BlockSpec((ROWBLK, HID), lambda i: (i, 0)),
        ],
        out_shape=[
            jax.ShapeDtypeStruct((NPAD, HID), jnp.float32),
            jax.ShapeDtypeStruct((NPAD, 1), jnp.float32),
            jax.ShapeDtypeStruct((NPAD, HID), jnp.float32),
        ],
    )(xs, W_enc, b_enc, W0, b0, degp)


def _k4_body(src_hbm, dst_hbm, g_hbm, zeros_hbm, out_hbm,
             dstidx, ring0, ring1, rows0, rows1,
             sem0, sem1, semi0, semi1, acc):
    c = lax.axis_index("c")
    s = lax.axis_index("s")
    wid = c * NS + s
    pltpu.sync_copy(zeros_hbm, acc.at[pl.ds(s * (NPAD // NS), NPAD // NS)])
    pltpu.sync_copy(dst_hbm.at[wid], dstidx)
    plsc.subcore_barrier()

    pltpu.sync_copy(src_hbm.at[wid, 0], ring0)
    pltpu.async_copy(g_hbm.at[ring0], rows0, sem0)
    pltpu.async_copy(src_hbm.at[wid, 1], ring1, semi1)

    def pair(p, _):
        i0 = 2 * p
        pltpu.make_async_copy(src_hbm.at[wid, 0], ring1, semi1).wait()
        pltpu.async_copy(g_hbm.at[ring1], rows1, sem1)
        pltpu.make_async_copy(g_hbm.at[ring0], rows0, sem0).wait()

        @pl.when(i0 + 2 < NCHUNK)
        def _():
            pltpu.async_copy(src_hbm.at[wid, i0 + 2], ring0, semi0)

        pltpu.sync_copy(rows0, acc.at[dstidx.at[i0]], add=True)

        @pl.when(i0 + 2 < NCHUNK)
        def _():
            pltpu.make_async_copy(src_hbm.at[wid, 0], ring0, semi0).wait()
            pltpu.async_copy(g_hbm.at[ring0], rows0, sem0)

        pltpu.make_async_copy(g_hbm.at[ring1], rows1, sem1).wait()
        pltpu.sync_copy(rows1, acc.at[dstidx.at[i0 + 1]], add=True)

        @pl.when(i0 + 3 < NCHUNK)
        def _():
            pltpu.async_copy(src_hbm.at[wid, i0 + 3], ring1, semi1)

        return 0

    lax.fori_loop(0, NPAIR, pair, 0)
    plsc.subcore_barrier()
    pltpu.sync_copy(acc.at[pl.ds(s * (NPAD // NS), NPAD // NS)], out_hbm.at[c, s])


def _k4(src2, dst2, g, zeros640):
    mesh = plsc.VectorSubcoreMesh(core_axis_name="c", subcore_axis_name="s")
    kfn = pl.kernel(
        _k4_body,
        out_type=jax.ShapeDtypeStruct((NC, NS, NPAD // NS, HID), jnp.float32),
        mesh=mesh,
        scratch_types=[
            pltpu.VMEM((NCHUNK, CHUNK), jnp.int32),
            pltpu.VMEM((CHUNK,), jnp.int32),
            pltpu.VMEM((CHUNK,), jnp.int32),
            pltpu.VMEM((CHUNK, HID), jnp.float32),
            pltpu.VMEM((CHUNK, HID), jnp.float32),
            pltpu.SemaphoreType.DMA,
            pltpu.SemaphoreType.DMA,
            pltpu.SemaphoreType.DMA,
            pltpu.SemaphoreType.DMA,
            pltpu.VMEM_SHARED((NPAD, HID), jnp.float32),
        ],
        compiler_params=pltpu.CompilerParams(needs_layout_passes=False),
    )
    return kfn(src2, dst2, g, zeros640)


def _k5_body(accp_ref, dinv_ref, zero_ref, w1_ref, b1_ref, out_ref):
    acc = accp_ref[0] + accp_ref[1]
    dv = dinv_ref[...]
    beta = LAMDA
    layer = (1.0 - beta) * (dv * acc) + beta * zero_ref[...]
    f = jnp.dot(layer, w1_ref[...], preferred_element_type=jnp.float32) + b1_ref[...]
    m = jnp.max(f, axis=1, keepdims=True)
    lse = jnp.log(jnp.sum(jnp.exp(f - m), axis=1, keepdims=True)) + m
    out_ref[...] = f - lse


def _k5(accp, dinv, zero_, W1, b1):
    return pl.pallas_call(
        _k5_body,
        grid=(NBLK,),
        in_specs=[
            pl.BlockSpec((NC, ROWBLK, HID), lambda i: (0, i, 0)),
            pl.BlockSpec((ROWBLK, 1), lambda i: (i, 0)),
            pl.BlockSpec((ROWBLK, HID), lambda i: (i, 0)),
            pl.BlockSpec((HID, NCLASS), lambda i: (0, 0)),
            pl.BlockSpec((1, NCLASS), lambda i: (0, 0)),
        ],
        out_specs=pl.BlockSpec((ROWBLK, NCLASS), lambda i: (i, 0)),
        out_shape=jax.ShapeDtypeStruct((NPAD, NCLASS), jnp.float32),
    )(accp, dinv, zero_, W1, b1)


@jax.jit
def kernel(x, edge_index, W_enc, b_enc, W0, b0, W1, b1):
    xs = jnp.squeeze(x, 0)
    xs = jnp.pad(xs, ((0, NPAD - N), (0, 0)))
    pad_idx = N + (jnp.arange(EPAD - E, dtype=jnp.int32) % (NPAD - N))
    ei = jnp.concatenate([edge_index, jnp.stack([pad_idx, pad_idx])], axis=1)
    src2 = ei[0].reshape(NW, NCHUNK, CHUNK)
    dst2 = ei[1].reshape(NW, NCHUNK, CHUNK)
    dst2h = ei[1].reshape(NW, EPP // 16, 16)
    b_enc2 = b_enc.reshape(1, HID)
    b02 = b0.reshape(1, HID)
    b12 = b1.reshape(1, NCLASS)

    zeros16 = jnp.zeros((NPAD,), jnp.float32)
    degp = _k2(dst2h, zeros16)
    degp = degp.reshape(NC, NPAD, 1)

    zero_, dinv, g = _k3(xs, W_enc, b_enc2, W0, b02, degp)

    zeros640 = jnp.zeros((NPAD // NS, HID), jnp.float32)
    accp = _k4(src2, dst2, g, zeros640)
    accp = accp.reshape(NC, NPAD, HID)

    logp = _k5(accp, dinv, zero_, W1, b12)[:N]
    return (logp, jnp.float32(0.0), 0, 0)


# speedup vs baseline: 1.2348x; 1.1414x over previous
import functools

import jax
import jax.numpy as jnp
from jax import lax
from jax.experimental import pallas as pl
from jax.experimental.pallas import tpu as pltpu
from jax.experimental.pallas import tpu_sc as plsc

N = 10000
NPAD = 10240
E = 320000
NFEAT = 128
HID = 128
NCLASS = 64
LAMDA = 0.5

NC = 2
NS = 16
NW = NC * NS
CHUNK = 120
NCHUNK = 84
EPP = NCHUNK * CHUNK
EPAD = NW * EPP
NTRI = NCHUNK // 3
ROWBLK = 2048
NBLK = NPAD // ROWBLK


STRIPE = NPAD // NS


def _k2_body(dst_hbm, zeros_hbm, out_hbm, dstv, hist, gath, sdeg):
    c = lax.axis_index("c")
    s = lax.axis_index("s")
    wid = c * NS + s
    pltpu.sync_copy(zeros_hbm, hist)
    pltpu.sync_copy(dst_hbm.at[wid], dstv)
    ones_v = jnp.ones((16,), jnp.float32)

    def step(i, _):
        d = dstv[i]
        plsc.addupdate_scatter(hist, [d], ones_v)
        return 0

    lax.fori_loop(0, EPP // 16, step, 0)
    pltpu.sync_copy(hist, sdeg.at[s])
    plsc.subcore_barrier()
    pltpu.sync_copy(sdeg.at[:, pl.ds(s * STRIPE, STRIPE)], gath)

    def red(j, _):
        sl = pl.ds(j * 16, 16)
        v = gath[0, sl]
        for t in range(1, NS):
            v = v + gath[t, sl]
        hist[sl] = v
        return 0

    lax.fori_loop(0, STRIPE // 16, red, 0)
    pltpu.sync_copy(hist.at[pl.ds(0, STRIPE)], out_hbm.at[c, pl.ds(s * STRIPE, STRIPE)])


def _k2(dst2, zeros16):
    mesh = plsc.VectorSubcoreMesh(core_axis_name="c", subcore_axis_name="s")
    kfn = pl.kernel(
        _k2_body,
        out_type=jax.ShapeDtypeStruct((NC, NPAD), jnp.float32),
        mesh=mesh,
        scratch_types=[
            pltpu.VMEM((EPP // 16, 16), jnp.int32),
            pltpu.VMEM((NPAD,), jnp.float32),
            pltpu.VMEM((NS, STRIPE), jnp.float32),
            pltpu.VMEM_SHARED((NS, NPAD), jnp.float32),
        ],
        compiler_params=pltpu.CompilerParams(needs_layout_passes=False),
    )
    return kfn(dst2, zeros16)


def _k3_body(x_ref, we_ref, be_ref, w0_ref, b0_ref, degp_ref,
             zero_ref, dinv_ref, g_ref):
    wc = jnp.dot(we_ref[...], w0_ref[...], preferred_element_type=jnp.float32)
    bc = jnp.dot(be_ref[...], w0_ref[...], preferred_element_type=jnp.float32) + b0_ref[...]
    zero = jnp.dot(x_ref[...], wc, preferred_element_type=jnp.float32) + bc
    zero_ref[...] = zero
    deg = jnp.sum(degp_ref[...], axis=0)
    dv = lax.rsqrt(jnp.maximum(deg, 1.0))
    dinv_ref[...] = dv
    g_ref[...] = dv * zero


def _k3(xs, W_enc, b_enc, W0, b0, degp):
    return pl.pallas_call(
        _k3_body,
        grid=(NBLK,),
        in_specs=[
            pl.BlockSpec((ROWBLK, NFEAT), lambda i: (i, 0)),
            pl.BlockSpec((NFEAT, HID), lambda i: (0, 0)),
            pl.BlockSpec((1, HID), lambda i: (0, 0)),
            pl.BlockSpec((HID, HID), lambda i: (0, 0)),
            pl.BlockSpec((1, HID), lambda i: (0, 0)),
            pl.BlockSpec((NC, ROWBLK, 1), lambda i: (0, i, 0)),
        ],
        out_specs=[
            pl.BlockSpec((ROWBLK, HID), lambda i: (i, 0)),
            pl.BlockSpec((ROWBLK, 1), lambda i: (i, 0)),
            pl.BlockSpec((ROWBLK, HID), lambda i: (i, 0)),
        ],
        out_shape=[
            jax.ShapeDtypeStruct((NPAD, HID), jnp.float32),
            jax.ShapeDtypeStruct((NPAD, 1), jnp.float32),
            jax.ShapeDtypeStruct((NPAD, HID), jnp.float32),
        ],
    )(xs, W_enc, b_enc, W0, b0, degp)


def _k4_body(src_hbm, dst_hbm, g_hbm, zeros_hbm, out_hbm,
             rs0, rs1, rs2, rd0, rd1, rd2, rows0, rows1, rows2,
             sem0, sem1, sem2, ss0, ss1, ss2, sd0, sd1, sd2, acc):
    c = lax.axis_index("c")
    s = lax.axis_index("s")
    wid = c * NS + s
    pltpu.sync_copy(zeros_hbm, acc.at[pl.ds(s * (NPAD // NS), NPAD // NS)])
    plsc.subcore_barrier()

    pltpu.async_copy(src_hbm.at[wid, 0], rs0, ss0)
    pltpu.async_copy(src_hbm.at[wid, 1], rs1, ss1)
    pltpu.async_copy(src_hbm.at[wid, 2], rs2, ss2)
    pltpu.async_copy(dst_hbm.at[wid, 0], rd0, sd0)
    pltpu.async_copy(dst_hbm.at[wid, 1], rd1, sd1)
    pltpu.async_copy(dst_hbm.at[wid, 2], rd2, sd2)
    pltpu.make_async_copy(src_hbm.at[wid, 0], rs0, ss0).wait()
    pltpu.async_copy(g_hbm.at[rs0], rows0, sem0)
    pltpu.make_async_copy(src_hbm.at[wid, 0], rs1, ss1).wait()
    pltpu.async_copy(g_hbm.at[rs1], rows1, sem1)

    def step(k, rs, rd, rows, sem, ss, sdm, rsn, rowsn, semn, ssn):
        @pl.when(k + 2 < NCHUNK)
        def _():
            pltpu.make_async_copy(src_hbm.at[wid, 0], rsn, ssn).wait()
            pltpu.async_copy(g_hbm.at[rsn], rowsn, semn)

        pltpu.make_async_copy(g_hbm.at[rs], rows, sem).wait()

        @pl.when(k + 3 < NCHUNK)
        def _():
            pltpu.async_copy(src_hbm.at[wid, k + 3], rs, ss)

        pltpu.make_async_copy(dst_hbm.at[wid, 0], rd, sdm).wait()
        pltpu.sync_copy(rows, acc.at[rd], add=True)

        @pl.when(k + 3 < NCHUNK)
        def _():
            pltpu.async_copy(dst_hbm.at[wid, k + 3], rd, sdm)

    def tri(t, _):
        j = 3 * t
        step(j, rs0, rd0, rows0, sem0, ss0, sd0, rs2, rows2, sem2, ss2)
        step(j + 1, rs1, rd1, rows1, sem1, ss1, sd1, rs0, rows0, sem0, ss0)
        step(j + 2, rs2, rd2, rows2, sem2, ss2, sd2, rs1, rows1, sem1, ss1)
        return 0

    lax.fori_loop(0, NTRI, tri, 0)
    plsc.subcore_barrier()
    pltpu.sync_copy(acc.at[pl.ds(s * (NPAD // NS), NPAD // NS)], out_hbm.at[c, s])


def _k4(src2, dst2, g, zeros640):
    mesh = plsc.VectorSubcoreMesh(core_axis_name="c", subcore_axis_name="s")
    kfn = pl.kernel(
        _k4_body,
        out_type=jax.ShapeDtypeStruct((NC, NS, NPAD // NS, HID), jnp.float32),
        mesh=mesh,
        scratch_types=[
            pltpu.VMEM((CHUNK,), jnp.int32),
            pltpu.VMEM((CHUNK,), jnp.int32),
            pltpu.VMEM((CHUNK,), jnp.int32),
            pltpu.VMEM((CHUNK,), jnp.int32),
            pltpu.VMEM((CHUNK,), jnp.int32),
            pltpu.VMEM((CHUNK,), jnp.int32),
            pltpu.VMEM((CHUNK, HID), jnp.float32),
            pltpu.VMEM((CHUNK, HID), jnp.float32),
            pltpu.VMEM((CHUNK, HID), jnp.float32),
            pltpu.SemaphoreType.DMA,
            pltpu.SemaphoreType.DMA,
            pltpu.SemaphoreType.DMA,
            pltpu.SemaphoreType.DMA,
            pltpu.SemaphoreType.DMA,
            pltpu.SemaphoreType.DMA,
            pltpu.SemaphoreType.DMA,
            pltpu.SemaphoreType.DMA,
            pltpu.SemaphoreType.DMA,
            pltpu.VMEM_SHARED((NPAD, HID), jnp.float32),
        ],
        compiler_params=pltpu.CompilerParams(needs_layout_passes=False),
    )
    return kfn(src2, dst2, g, zeros640)


def _k5_body(accp_ref, dinv_ref, zero_ref, w1_ref, b1_ref, out_ref):
    acc = accp_ref[0] + accp_ref[1]
    dv = dinv_ref[...]
    beta = LAMDA
    layer = (1.0 - beta) * (dv * acc) + beta * zero_ref[...]
    f = jnp.dot(layer, w1_ref[...], preferred_element_type=jnp.float32) + b1_ref[...]
    m = jnp.max(f, axis=1, keepdims=True)
    lse = jnp.log(jnp.sum(jnp.exp(f - m), axis=1, keepdims=True)) + m
    out_ref[...] = f - lse


def _k5(accp, dinv, zero_, W1, b1):
    return pl.pallas_call(
        _k5_body,
        grid=(NBLK,),
        in_specs=[
            pl.BlockSpec((NC, ROWBLK, HID), lambda i: (0, i, 0)),
            pl.BlockSpec((ROWBLK, 1), lambda i: (i, 0)),
            pl.BlockSpec((ROWBLK, HID), lambda i: (i, 0)),
            pl.BlockSpec((HID, NCLASS), lambda i: (0, 0)),
            pl.BlockSpec((1, NCLASS), lambda i: (0, 0)),
        ],
        out_specs=pl.BlockSpec((ROWBLK, NCLASS), lambda i: (i, 0)),
        out_shape=jax.ShapeDtypeStruct((NPAD, NCLASS), jnp.float32),
    )(accp, dinv, zero_, W1, b1)


@jax.jit
def kernel(x, edge_index, W_enc, b_enc, W0, b0, W1, b1):
    xs = jnp.squeeze(x, 0)
    xs = jnp.pad(xs, ((0, NPAD - N), (0, 0)))
    pad_idx = N + (jnp.arange(EPAD - E, dtype=jnp.int32) % (NPAD - N))
    ei = jnp.concatenate([edge_index, jnp.stack([pad_idx, pad_idx])], axis=1)
    src2 = ei[0].reshape(NW, NCHUNK, CHUNK)
    dst2 = ei[1].reshape(NW, NCHUNK, CHUNK)
    dst2h = ei[1].reshape(NW, EPP // 16, 16)
    b_enc2 = b_enc.reshape(1, HID)
    b02 = b0.reshape(1, HID)
    b12 = b1.reshape(1, NCLASS)

    zeros16 = jnp.zeros((NPAD,), jnp.float32)
    degp = _k2(dst2h, zeros16)
    degp = degp.reshape(NC, NPAD, 1)

    zero_, dinv, g = _k3(xs, W_enc, b_enc2, W0, b02, degp)

    zeros640 = jnp.zeros((NPAD // NS, HID), jnp.float32)
    accp = _k4(src2, dst2, g, zeros640)
    accp = accp.reshape(NC, NPAD, HID)

    logp = _k5(accp, dinv, zero_, W1, b12)[:N]
    return (logp, jnp.float32(0.0), 0, 0)


# speedup vs baseline: 1.2351x; 1.0002x over previous
import jax
import jax.numpy as jnp
from jax import lax
from jax.experimental import pallas as pl
from jax.experimental.pallas import tpu as pltpu
from jax.experimental.pallas import tpu_sc as plsc

N = 10000
NPAD = 10240
E = 320000
NFEAT = 128
HID = 128
NCLASS = 64
LAMDA = 0.5

NC = 2
NS = 16
NW = NC * NS
CHUNK = 120
NCHUNK = 84
EPP = NCHUNK * CHUNK
EPAD = NW * EPP
NTRI = NCHUNK // 3
ROWBLK = 2048
NBLK = NPAD // ROWBLK


STRIPE = NPAD // NS


def _k2_body(dst_hbm, zeros_hbm, out_hbm, dstv, hist, gath, sdeg):
    c = lax.axis_index("c")
    s = lax.axis_index("s")
    wid = c * NS + s
    pltpu.sync_copy(zeros_hbm, hist)
    pltpu.sync_copy(dst_hbm.at[wid], dstv)
    ones_v = jnp.ones((16,), jnp.float32)

    def step(i, _):
        d = dstv[i]
        plsc.addupdate_scatter(hist, [d], ones_v)
        return 0

    lax.fori_loop(0, EPP // 16, step, 0)
    pltpu.sync_copy(hist, sdeg.at[s])
    plsc.subcore_barrier()
    pltpu.sync_copy(sdeg.at[:, pl.ds(s * STRIPE, STRIPE)], gath)

    def red(j, _):
        sl = pl.ds(j * 16, 16)
        v = gath[0, sl]
        for t in range(1, NS):
            v = v + gath[t, sl]
        hist[sl] = v
        return 0

    lax.fori_loop(0, STRIPE // 16, red, 0)
    pltpu.sync_copy(hist.at[pl.ds(0, STRIPE)], out_hbm.at[c, pl.ds(s * STRIPE, STRIPE)])


def _k2(dst2, zeros16):
    mesh = plsc.VectorSubcoreMesh(core_axis_name="c", subcore_axis_name="s")
    kfn = pl.kernel(
        _k2_body,
        out_type=jax.ShapeDtypeStruct((NC, NPAD), jnp.float32),
        mesh=mesh,
        scratch_types=[
            pltpu.VMEM((EPP // 16, 16), jnp.int32),
            pltpu.VMEM((NPAD,), jnp.float32),
            pltpu.VMEM((NS, STRIPE), jnp.float32),
            pltpu.VMEM_SHARED((NS, NPAD), jnp.float32),
        ],
        compiler_params=pltpu.CompilerParams(needs_layout_passes=False),
    )
    return kfn(dst2, zeros16)


def _k3_body(x_ref, we_ref, be_ref, w0_ref, b0_ref, degp_ref,
             zero_ref, dinv_ref, g_ref):
    wc = jnp.dot(we_ref[...], w0_ref[...], preferred_element_type=jnp.float32)
    bc = jnp.dot(be_ref[...], w0_ref[...], preferred_element_type=jnp.float32) + b0_ref[...]
    zero = jnp.dot(x_ref[...], wc, preferred_element_type=jnp.float32) + bc
    zero_ref[...] = zero
    deg = jnp.sum(degp_ref[...], axis=0)
    dv = lax.rsqrt(jnp.maximum(deg, 1.0))
    dinv_ref[...] = dv
    g_ref[...] = dv * zero


def _k3(xs, W_enc, b_enc, W0, b0, degp):
    return pl.pallas_call(
        _k3_body,
        grid=(NBLK,),
        in_specs=[
            pl.BlockSpec((ROWBLK, NFEAT), lambda i: (i, 0)),
            pl.BlockSpec((NFEAT, HID), lambda i: (0, 0)),
            pl.BlockSpec((1, HID), lambda i: (0, 0)),
            pl.BlockSpec((HID, HID), lambda i: (0, 0)),
            pl.BlockSpec((1, HID), lambda i: (0, 0)),
            pl.BlockSpec((NC, ROWBLK, 1), lambda i: (0, i, 0)),
        ],
        out_specs=[
            pl.BlockSpec((ROWBLK, HID), lambda i: (i, 0)),
            pl.BlockSpec((ROWBLK, 1), lambda i: (i, 0)),
            pl.BlockSpec((ROWBLK, HID), lambda i: (i, 0)),
        ],
        out_shape=[
            jax.ShapeDtypeStruct((NPAD, HID), jnp.float32),
            jax.ShapeDtypeStruct((NPAD, 1), jnp.float32),
            jax.ShapeDtypeStruct((NPAD, HID), jnp.float32),
        ],
    )(xs, W_enc, b_enc, W0, b0, degp)


def _k4_body(src_hbm, dst_hbm, g_hbm, zeros_hbm, out_hbm,
             rs0, rs1, rs2, rd0, rd1, rd2, rows0, rows1, rows2,
             sem0, sem1, sem2, ss0, ss1, ss2, sd0, sd1, sd2, acc):
    c = lax.axis_index("c")
    s = lax.axis_index("s")
    wid = c * NS + s
    pltpu.sync_copy(zeros_hbm, acc.at[pl.ds(s * (NPAD // NS), NPAD // NS)])
    plsc.subcore_barrier()

    pltpu.async_copy(src_hbm.at[wid, 0], rs0, ss0)
    pltpu.async_copy(src_hbm.at[wid, 1], rs1, ss1)
    pltpu.async_copy(src_hbm.at[wid, 2], rs2, ss2)
    pltpu.async_copy(dst_hbm.at[wid, 0], rd0, sd0)
    pltpu.async_copy(dst_hbm.at[wid, 1], rd1, sd1)
    pltpu.async_copy(dst_hbm.at[wid, 2], rd2, sd2)
    pltpu.make_async_copy(src_hbm.at[wid, 0], rs0, ss0).wait()
    pltpu.async_copy(g_hbm.at[rs0], rows0, sem0)
    pltpu.make_async_copy(src_hbm.at[wid, 0], rs1, ss1).wait()
    pltpu.async_copy(g_hbm.at[rs1], rows1, sem1)

    def step(k, rs, rd, rows, sem, ss, sdm, rsn, rowsn, semn, ssn):
        @pl.when(k + 2 < NCHUNK)
        def _():
            pltpu.make_async_copy(src_hbm.at[wid, 0], rsn, ssn).wait()
            pltpu.async_copy(g_hbm.at[rsn], rowsn, semn)

        pltpu.make_async_copy(g_hbm.at[rs], rows, sem).wait()

        @pl.when(k + 3 < NCHUNK)
        def _():
            pltpu.async_copy(src_hbm.at[wid, k + 3], rs, ss)

        pltpu.make_async_copy(dst_hbm.at[wid, 0], rd, sdm).wait()
        pltpu.sync_copy(rows, acc.at[rd], add=True)

        @pl.when(k + 3 < NCHUNK)
        def _():
            pltpu.async_copy(dst_hbm.at[wid, k + 3], rd, sdm)

    def tri(t, _):
        j = 3 * t
        step(j, rs0, rd0, rows0, sem0, ss0, sd0, rs2, rows2, sem2, ss2)
        step(j + 1, rs1, rd1, rows1, sem1, ss1, sd1, rs0, rows0, sem0, ss0)
        step(j + 2, rs2, rd2, rows2, sem2, ss2, sd2, rs1, rows1, sem1, ss1)
        return 0

    lax.fori_loop(0, NTRI, tri, 0)
    plsc.subcore_barrier()
    pltpu.sync_copy(acc.at[pl.ds(s * (NPAD // NS), NPAD // NS)], out_hbm.at[c, s])


def _k4(src2, dst2, g, zeros640):
    mesh = plsc.VectorSubcoreMesh(core_axis_name="c", subcore_axis_name="s")
    kfn = pl.kernel(
        _k4_body,
        out_type=jax.ShapeDtypeStruct((NC, NS, NPAD // NS, HID), jnp.float32),
        mesh=mesh,
        scratch_types=[
            pltpu.VMEM((CHUNK,), jnp.int32),
            pltpu.VMEM((CHUNK,), jnp.int32),
            pltpu.VMEM((CHUNK,), jnp.int32),
            pltpu.VMEM((CHUNK,), jnp.int32),
            pltpu.VMEM((CHUNK,), jnp.int32),
            pltpu.VMEM((CHUNK,), jnp.int32),
            pltpu.VMEM((CHUNK, HID), jnp.float32),
            pltpu.VMEM((CHUNK, HID), jnp.float32),
            pltpu.VMEM((CHUNK, HID), jnp.float32),
            pltpu.SemaphoreType.DMA,
            pltpu.SemaphoreType.DMA,
            pltpu.SemaphoreType.DMA,
            pltpu.SemaphoreType.DMA,
            pltpu.SemaphoreType.DMA,
            pltpu.SemaphoreType.DMA,
            pltpu.SemaphoreType.DMA,
            pltpu.SemaphoreType.DMA,
            pltpu.SemaphoreType.DMA,
            pltpu.VMEM_SHARED((NPAD, HID), jnp.float32),
        ],
        compiler_params=pltpu.CompilerParams(needs_layout_passes=False),
    )
    return kfn(src2, dst2, g, zeros640)


def _k5_body(accp_ref, dinv_ref, zero_ref, w1_ref, b1_ref, out_ref):
    acc = accp_ref[0] + accp_ref[1]
    dv = dinv_ref[...]
    beta = LAMDA
    layer = (1.0 - beta) * (dv * acc) + beta * zero_ref[...]
    f = jnp.dot(layer, w1_ref[...], preferred_element_type=jnp.float32) + b1_ref[...]
    m = jnp.max(f, axis=1, keepdims=True)
    lse = jnp.log(jnp.sum(jnp.exp(f - m), axis=1, keepdims=True)) + m
    out_ref[...] = f - lse


def _k5(accp, dinv, zero_, W1, b1):
    return pl.pallas_call(
        _k5_body,
        grid=(NBLK,),
        in_specs=[
            pl.BlockSpec((NC, ROWBLK, HID), lambda i: (0, i, 0)),
            pl.BlockSpec((ROWBLK, 1), lambda i: (i, 0)),
            pl.BlockSpec((ROWBLK, HID), lambda i: (i, 0)),
            pl.BlockSpec((HID, NCLASS), lambda i: (0, 0)),
            pl.BlockSpec((1, NCLASS), lambda i: (0, 0)),
        ],
        out_specs=pl.BlockSpec((ROWBLK, NCLASS), lambda i: (i, 0)),
        out_shape=jax.ShapeDtypeStruct((NPAD, NCLASS), jnp.float32),
    )(accp, dinv, zero_, W1, b1)


@jax.jit
def kernel(x, edge_index, W_enc, b_enc, W0, b0, W1, b1):
    xs = jnp.squeeze(x, 0)
    xs = jnp.pad(xs, ((0, NPAD - N), (0, 0)))
    pad_idx = N + (jnp.arange(EPAD - E, dtype=jnp.int32) % (NPAD - N))
    ei = jnp.concatenate([edge_index, jnp.stack([pad_idx, pad_idx])], axis=1)
    src2 = ei[0].reshape(NW, NCHUNK, CHUNK)
    dst2 = ei[1].reshape(NW, NCHUNK, CHUNK)
    dst2h = ei[1].reshape(NW, EPP // 16, 16)
    b_enc2 = b_enc.reshape(1, HID)
    b02 = b0.reshape(1, HID)
    b12 = b1.reshape(1, NCLASS)

    zeros16 = jnp.zeros((NPAD,), jnp.float32)
    degp = _k2(dst2h, zeros16)
    degp = degp.reshape(NC, NPAD, 1)

    zero_, dinv, g = _k3(xs, W_enc, b_enc2, W0, b02, degp)

    zeros640 = jnp.zeros((NPAD // NS, HID), jnp.float32)
    accp = _k4(src2, dst2, g, zeros640)
    accp = accp.reshape(NC, NPAD, HID)

    logp = _k5(accp, dinv, zero_, W1, b12)[:N]
    return (logp, jnp.float32(0.0), 0, 0)
